# baseline (speedup 1.0000x reference)
import functools

import jax
import jax.numpy as jnp
from jax import lax
from jax.experimental import pallas as pl
from jax.experimental.pallas import tpu as pltpu
from jax.experimental.pallas import tpu_sc as plsc

B, S, H = 2, 2048, 1024
E, TOPK = 8, 2
I_MOE = 1024
I_SH = 2816
N = B * S
P = N * TOPK

BM = 256
CAP = P + E * BM
NB = CAP // BM

BMR = 256
NCHUNK = P // BMR
BMS = 512
BIS = 1408
NIS = I_SH // BIS

NTILE = 32
TPT = N // NTILE
CSUB = 32


def _router_body(x_ref, gw_ref, a1_ref, a2_ref, w1_ref, w2_ref,
                 aux_ref, c1_ref, c2_ref, z_ref, psum_ref, cnt_ref):
    step = pl.program_id(0)
    x = x_ref[...]
    gw = gw_ref[...]
    lg = lax.dot_general(x, gw, (((1,), (1,)), ((), ())),
                         preferred_element_type=jnp.float32)
    z_ref[...] = lg[:, E:E + 1]
    logits = lg[:, :E]
    ii = lax.broadcasted_iota(jnp.int32, logits.shape, 1)
    m1 = jnp.max(logits, axis=1, keepdims=True)
    a1 = jnp.min(jnp.where(logits >= m1, ii, E), axis=1, keepdims=True)
    l2 = jnp.where(ii == a1, -jnp.inf, logits)
    m2 = jnp.max(l2, axis=1, keepdims=True)
    a2 = jnp.min(jnp.where(l2 >= m2, ii, E), axis=1, keepdims=True)
    ex = jnp.exp(logits - m1)
    s = jnp.sum(ex, axis=1, keepdims=True)
    a1_ref[...] = a1
    a2_ref[...] = a2
    w1_ref[...] = 1.0 / s
    w2_ref[...] = jnp.exp(m2 - m1) / s

    probs = ex / s
    oh1 = (ii == a1).astype(jnp.float32)
    oh2 = (ii == a2).astype(jnp.float32)
    c1 = jnp.sum(oh1, axis=0, keepdims=True)
    c2 = jnp.sum(oh2, axis=0, keepdims=True)
    c1_ref[...] = c1.astype(jnp.int32)[None]
    c2_ref[...] = c2.astype(jnp.int32)[None]

    @pl.when(step == 0)
    def _init():
        psum_ref[...] = jnp.zeros_like(psum_ref)
        cnt_ref[...] = jnp.zeros_like(cnt_ref)

    psum_ref[...] += jnp.sum(probs, axis=0, keepdims=True)
    cnt_ref[...] += c1 + c2

    @pl.when(step == pl.num_programs(0) - 1)
    def _fin():
        frac_tok = cnt_ref[...] / float(N * TOPK)
        frac_prob = psum_ref[...] / float(N)
        aux_ref[0, 0] = float(E) * jnp.sum(frac_tok * frac_prob)


def _router(x, gate_weight, shared_gate_w):
    grid = (N // BMR,)
    return pl.pallas_call(
        _router_body,
        grid=grid,
        in_specs=[
            pl.BlockSpec((BMR, H), lambda i: (i, 0)),
            pl.BlockSpec((E + 1, H), lambda i: (0, 0)),
        ],
        out_specs=[
            pl.BlockSpec((BMR, 1), lambda i: (i, 0)),
            pl.BlockSpec((BMR, 1), lambda i: (i, 0)),
            pl.BlockSpec((BMR, 1), lambda i: (i, 0)),
            pl.BlockSpec((BMR, 1), lambda i: (i, 0)),
            pl.BlockSpec(memory_space=pltpu.SMEM),
            pl.BlockSpec((1, 1, E), lambda i: (i, 0, 0)),
            pl.BlockSpec((1, 1, E), lambda i: (i, 0, 0)),
            pl.BlockSpec((BMR, 1), lambda i: (i, 0)),
        ],
        out_shape=[
            jax.ShapeDtypeStruct((N, 1), jnp.int32),
            jax.ShapeDtypeStruct((N, 1), jnp.int32),
            jax.ShapeDtypeStruct((N, 1), jnp.float32),
            jax.ShapeDtypeStruct((N, 1), jnp.float32),
            jax.ShapeDtypeStruct((1, 1), jnp.float32),
            jax.ShapeDtypeStruct((NCHUNK // 2, 1, E), jnp.int32),
            jax.ShapeDtypeStruct((NCHUNK // 2, 1, E), jnp.int32),
            jax.ShapeDtypeStruct((N, 1), jnp.float32),
        ],
        scratch_shapes=[
            pltpu.VMEM((1, E), jnp.float32),
            pltpu.VMEM((1, E), jnp.float32),
        ],
    )(x, jnp.concatenate([gate_weight, shared_gate_w], axis=0))


def _dispatch_body(eid_hbm, bases_hbm, x_hbm,
                   xs_hbm, pos_hbm,
                   eid_v, dest_v, bases_v, buf0, buf1, sem0, sem1):
    w = lax.axis_index("s") * 2 + lax.axis_index("c")
    base = pl.multiple_of(w * BMR, BMR)
    pltpu.sync_copy(eid_hbm.at[pl.ds(base, BMR)], eid_v)
    pltpu.sync_copy(bases_hbm.at[w], bases_v)

    lanes = lax.broadcasted_iota(jnp.int32, (16,), 0)
    dnums = lax.GatherDimensionNumbers(
        offset_dims=(), collapsed_slice_dims=(0,), start_index_map=(0,))

    def bcast_lane(vec, e):
        idx = jnp.full((16, 1), e, jnp.int32)
        return lax.gather(vec, idx, dnums, (1,),
                          mode=lax.GatherScatterMode.PROMISE_IN_BOUNDS)

    bv = bases_v[...]
    for i in range(BMR // 16):
        v = eid_v[pl.ds(i * 16, 16)]
        dest = jnp.zeros((16,), jnp.int32)
        for e in range(E):
            m = v == e
            csum = plsc.cumsum(jnp.where(m, 1, 0).astype(jnp.int32))
            dest = jnp.where(m, bcast_lane(bv, e) + csum - 1, dest)
            pc = plsc.all_reduce_population_count(m)
            bv = bv + jnp.where(lanes == e, pc, 0)
        dest_v[i // 2, pl.ds((i % 2) * 16, 16)] = dest

    pltpu.sync_copy(dest_v, pos_hbm.at[w])

    tstart = pl.multiple_of((w % (NCHUNK // 2)) * BMR, BMR)
    bufs = (buf0, buf1)
    sems = (sem0, sem1)
    descs = [None, None]
    for c in range(8):
        if descs[c % 2] is not None:
            descs[c % 2].wait()
        pltpu.sync_copy(x_hbm.at[pl.ds(tstart + c * 32, 32)], bufs[c % 2])
        descs[c % 2] = pltpu.async_copy(
            bufs[c % 2], xs_hbm.at[dest_v.at[c]], sems[c % 2])
    descs[0].wait()
    descs[1].wait()


def _dispatch(eid, bases, x):
    mesh = plsc.VectorSubcoreMesh(core_axis_name="c", subcore_axis_name="s")
    f = functools.partial(
        pl.kernel,
        mesh=mesh,
        compiler_params=pltpu.CompilerParams(needs_layout_passes=False),
        out_type=[
            jax.ShapeDtypeStruct((CAP, H), jnp.float32),
            jax.ShapeDtypeStruct((NCHUNK, 8, 32), jnp.int32),
        ],
        scratch_types=[
            pltpu.VMEM((BMR,), jnp.int32),
            pltpu.VMEM((8, 32), jnp.int32),
            pltpu.VMEM((16,), jnp.int32),
            pltpu.VMEM((32, H), jnp.float32),
            pltpu.VMEM((32, H), jnp.float32),
            pltpu.SemaphoreType.DMA,
            pltpu.SemaphoreType.DMA,
        ],
    )(_dispatch_body)
    return f(eid, bases, x)


def _ffn_body(be_ref, nb_ref, xs_ref, gu_ref, dn_ref, ys_ref):
    b = pl.program_id(0)

    @pl.when(b < nb_ref[0])
    def _():
        x = xs_ref[...].astype(jnp.bfloat16)
        gu = gu_ref[0].astype(jnp.bfloat16)
        gup = lax.dot_general(x, gu, (((1,), (1,)), ((), ())),
                              preferred_element_type=jnp.float32)
        g = gup[:, :I_MOE]
        u = gup[:, I_MOE:]
        h = (g * jax.nn.sigmoid(g) * u).astype(jnp.bfloat16)
        dn = dn_ref[0].astype(jnp.bfloat16)
        ys_ref[...] = lax.dot_general(h, dn, (((1,), (1,)), ((), ())),
                                      preferred_element_type=jnp.float32)


def _grouped_ffn(block_expert, nblocks, xs, gate_up_proj, down_proj):
    grid_spec = pltpu.PrefetchScalarGridSpec(
        num_scalar_prefetch=2,
        grid=(NB,),
        in_specs=[
            pl.BlockSpec((BM, H), lambda b, be, nb: (b, 0)),
            pl.BlockSpec((1, 2 * I_MOE, H), lambda b, be, nb: (be[b], 0, 0)),
            pl.BlockSpec((1, H, I_MOE), lambda b, be, nb: (be[b], 0, 0)),
        ],
        out_specs=pl.BlockSpec((BM, H), lambda b, be, nb: (b, 0)),
    )
    return pl.pallas_call(
        _ffn_body,
        grid_spec=grid_spec,
        out_shape=jax.ShapeDtypeStruct((CAP, H), jnp.float32),
    )(block_expert, nblocks, xs, gate_up_proj, down_proj)


def _shared_half_body(x_ref, g_ref, u_ref, d_ref, out_ref):
    x = x_ref[...]
    gw = g_ref[...].astype(jnp.bfloat16)
    uw = u_ref[...].astype(jnp.bfloat16)
    g = lax.dot_general(x, gw, (((1,), (1,)), ((), ())),
                        preferred_element_type=jnp.float32)
    u = lax.dot_general(x, uw, (((1,), (1,)), ((), ())),
                        preferred_element_type=jnp.float32)
    h = (g * jax.nn.sigmoid(g) * u).astype(jnp.bfloat16)
    dw = d_ref[...].astype(jnp.bfloat16)
    out_ref[...] = lax.dot_general(h, dw, (((1,), (1,)), ((), ())),
                                   preferred_element_type=jnp.float32)


def _shared_final_body(x_ref, g_ref, u_ref, d_ref, sh0_ref, z_ref, ymoe_ref,
                       prev_ref, out_ref):
    del prev_ref
    x = x_ref[...]
    gw = g_ref[...].astype(jnp.bfloat16)
    uw = u_ref[...].astype(jnp.bfloat16)
    g = lax.dot_general(x, gw, (((1,), (1,)), ((), ())),
                        preferred_element_type=jnp.float32)
    u = lax.dot_general(x, uw, (((1,), (1,)), ((), ())),
                        preferred_element_type=jnp.float32)
    h = (g * jax.nn.sigmoid(g) * u).astype(jnp.bfloat16)
    dw = d_ref[...].astype(jnp.bfloat16)
    contrib = lax.dot_general(h, dw, (((1,), (1,)), ((), ())),
                              preferred_element_type=jnp.float32)
    out_ref[...] = ((sh0_ref[...] + contrib) * jax.nn.sigmoid(z_ref[...])
                    + ymoe_ref[...])


NH = N // 2
SBH = NH // BMS


def _shared_half(xb, sh_gate, sh_up, sh_down, half):
    return pl.pallas_call(
        _shared_half_body,
        grid=(SBH,),
        in_specs=[
            pl.BlockSpec((BMS, H), lambda m: (m + half * SBH, 0)),
            pl.BlockSpec((BIS, H), lambda m: (0, 0)),
            pl.BlockSpec((BIS, H), lambda m: (0, 0)),
            pl.BlockSpec((H, BIS), lambda m: (0, 0)),
        ],
        out_specs=pl.BlockSpec((BMS, H), lambda m: (m, 0)),
        out_shape=jax.ShapeDtypeStruct((NH, H), jnp.float32),
    )(xb, sh_gate, sh_up, sh_down)


def _shared_final(xb, sh_gate, sh_up, sh_down, sh0, z, ymoe, prev, half):
    in_specs = [
        pl.BlockSpec((BMS, H), lambda m: (m + half * SBH, 0)),
        pl.BlockSpec((BIS, H), lambda m: (1, 0)),
        pl.BlockSpec((BIS, H), lambda m: (1, 0)),
        pl.BlockSpec((H, BIS), lambda m: (0, 1)),
        pl.BlockSpec((BMS, H), lambda m: (m, 0)),
        pl.BlockSpec((BMS, 1), lambda m: (m + half * SBH, 0)),
        pl.BlockSpec((BMS, H), lambda m: (m, 0)),
    ]
    args = [xb, sh_gate, sh_up, sh_down, sh0, z, ymoe]
    aliases = {}
    body = _shared_final_body
    if prev is not None:
        in_specs.append(pl.BlockSpec(memory_space=pl.ANY))
        args.append(prev)
        aliases = {7: 0}
    else:
        def body(x_ref, g_ref, u_ref, d_ref, sh0_ref, z_ref, ymoe_ref,
                 out_ref):
            _shared_final_body(x_ref, g_ref, u_ref, d_ref, sh0_ref, z_ref,
                               ymoe_ref, None, out_ref)
    return pl.pallas_call(
        body,
        grid=(SBH,),
        in_specs=in_specs,
        out_specs=pl.BlockSpec((BMS, H), lambda m: (m + half * SBH, 0)),
        out_shape=jax.ShapeDtypeStruct((N, H), jnp.float32),
        input_output_aliases=aliases,
    )(*args)


def _combine_body(half, ys_hbm, pos_hbm, w1_hbm, w2_hbm, out_hbm,
                  idx1, idx2, wv1, wv2, y1, y2, acc, sem1, sem2):
    w = lax.axis_index("s") * 2 + lax.axis_index("c")
    tpt = NH // NTILE
    for sub in range(tpt // CSUB):
        lt0 = pl.multiple_of(w * tpt + sub * CSUB, CSUB)
        t0 = pl.multiple_of(half * NH + w * tpt + sub * CSUB, CSUB)
        pltpu.sync_copy(pos_hbm.at[pl.ds(t0, CSUB)], idx1)
        pltpu.sync_copy(pos_hbm.at[pl.ds(N + t0, CSUB)], idx2)
        d1 = pltpu.async_copy(ys_hbm.at[idx1], y1, sem1)
        d2 = pltpu.async_copy(ys_hbm.at[idx2], y2, sem2)
        pltpu.sync_copy(w1_hbm.at[pl.ds(t0, CSUB)], wv1)
        pltpu.sync_copy(w2_hbm.at[pl.ds(t0, CSUB)], wv2)
        d1.wait()
        d2.wait()

        def row(r, _):
            ri = jnp.full((16,), r, jnp.int32)
            b1 = plsc.load_gather(wv1, [ri])
            b2 = plsc.load_gather(wv2, [ri])
            for j in range(H // 16):
                sl = pl.ds(j * 16, 16)
                acc[r, sl] = b1 * y1[r, sl] + b2 * y2[r, sl]
            return 0

        lax.fori_loop(0, CSUB, row, 0)
        pltpu.sync_copy(acc, out_hbm.at[pl.ds(lt0, CSUB)])


def _combine(ys, pos_flat, w1, w2, half):
    mesh = plsc.VectorSubcoreMesh(core_axis_name="c", subcore_axis_name="s")
    f = functools.partial(
        pl.kernel,
        mesh=mesh,
        compiler_params=pltpu.CompilerParams(needs_layout_passes=False),
        out_type=jax.ShapeDtypeStruct((NH, H), jnp.float32),
        scratch_types=[
            pltpu.VMEM((CSUB,), jnp.int32),
            pltpu.VMEM((CSUB,), jnp.int32),
            pltpu.VMEM((CSUB,), jnp.float32),
            pltpu.VMEM((CSUB,), jnp.float32),
            pltpu.VMEM((CSUB, H), jnp.float32),
            pltpu.VMEM((CSUB, H), jnp.float32),
            pltpu.VMEM((CSUB, H), jnp.float32),
            pltpu.SemaphoreType.DMA,
            pltpu.SemaphoreType.DMA,
        ],
    )(functools.partial(_combine_body, half))
    return f(ys, pos_flat, w1, w2)


def kernel(hidden_states, gate_weight, gate_up_proj, down_proj,
           sh_gate_proj, sh_up_proj, sh_down_proj, shared_gate_w):
    x = hidden_states.reshape(N, H)

    a1, a2, w1, w2, aux, c1, c2, z = _router(x, gate_weight, shared_gate_w)

    cnts = jnp.concatenate([c1.reshape(NCHUNK // 2, E),
                            c2.reshape(NCHUNK // 2, E)], axis=0)
    counts = jnp.sum(cnts, axis=0)
    padded = ((counts + BM - 1) // BM) * BM
    poff = jnp.concatenate([jnp.zeros((1,), jnp.int32),
                            jnp.cumsum(padded)]).astype(jnp.int32)
    tilebase = poff[:E][None, :] + jnp.cumsum(cnts, axis=0) - cnts
    bases = jnp.pad(tilebase, ((0, 0), (0, 16 - E))).astype(jnp.int32)
    nblocks = (poff[E] // BM).reshape(1)
    bidx = jnp.arange(NB, dtype=jnp.int32) * BM
    block_expert = jnp.sum(
        (poff[1:E + 1][None, :] <= bidx[:, None]).astype(jnp.int32), axis=1)
    block_expert = jnp.minimum(block_expert, E - 1)

    eid = jnp.concatenate([a1[:, 0], a2[:, 0]])
    xb = x.astype(jnp.bfloat16)

    xs, pos = _dispatch(eid, bases, x)
    ys = _grouped_ffn(block_expert, nblocks, xs, gate_up_proj, down_proj)

    pos_flat = pos.reshape(P)
    w1f, w2f = w1.reshape(N), w2.reshape(N)
    sh0a = _shared_half(xb, sh_gate_proj, sh_up_proj, sh_down_proj, 0)
    sh0b = _shared_half(xb, sh_gate_proj, sh_up_proj, sh_down_proj, 1)
    ymoe_a = _combine(ys, pos_flat, w1f, w2f, 0)
    ymoe_b = _combine(ys, pos_flat, w1f, w2f, 1)
    out = _shared_final(xb, sh_gate_proj, sh_up_proj, sh_down_proj,
                        sh0a, z, ymoe_a, None, 0)
    out = _shared_final(xb, sh_gate_proj, sh_up_proj, sh_down_proj,
                        sh0b, z, ymoe_b, out, 1)
    return out.reshape(B, S, H), aux[0, 0]


# speedup vs baseline: 1.1109x; 1.1109x over previous
import functools

import jax
import jax.numpy as jnp
from jax import lax
from jax.experimental import pallas as pl
from jax.experimental.pallas import tpu as pltpu
from jax.experimental.pallas import tpu_sc as plsc

B, S, H = 2, 2048, 1024
E, TOPK = 8, 2
I_MOE = 1024
I_SH = 2816
N = B * S
P = N * TOPK

BM = 256
CAP = P + E * BM
NB = CAP // BM

BMR = 256
NCHUNK = P // BMR
BMS = 512
BIS = 1408
NIS = I_SH // BIS

NTILE = 32
TPT = N // NTILE
CSUB = 32


def _router_body(x_ref, gw_ref, sgw_ref, a1_ref, a2_ref, w1_ref, w2_ref,
                 aux_ref, c1_ref, c2_ref, z_ref, psum_ref, cnt_ref):
    step = pl.program_id(0)
    x = x_ref[...]
    z_ref[...] = lax.dot_general(x, sgw_ref[...], (((1,), (1,)), ((), ())),
                                 preferred_element_type=jnp.float32)
    logits = lax.dot_general(x, gw_ref[...], (((1,), (1,)), ((), ())),
                             preferred_element_type=jnp.float32)
    ii = lax.broadcasted_iota(jnp.int32, logits.shape, 1)
    m1 = jnp.max(logits, axis=1, keepdims=True)
    a1 = jnp.min(jnp.where(logits >= m1, ii, E), axis=1, keepdims=True)
    l2 = jnp.where(ii == a1, -jnp.inf, logits)
    m2 = jnp.max(l2, axis=1, keepdims=True)
    a2 = jnp.min(jnp.where(l2 >= m2, ii, E), axis=1, keepdims=True)
    ex = jnp.exp(logits - m1)
    s = jnp.sum(ex, axis=1, keepdims=True)
    a1_ref[...] = a1
    a2_ref[...] = a2
    w1_ref[...] = 1.0 / s
    w2_ref[...] = jnp.exp(m2 - m1) / s

    probs = ex / s
    oh1 = (ii == a1).astype(jnp.float32)
    oh2 = (ii == a2).astype(jnp.float32)
    c1 = jnp.sum(oh1, axis=0, keepdims=True)
    c2 = jnp.sum(oh2, axis=0, keepdims=True)
    c1_ref[...] = c1.astype(jnp.int32)[None]
    c2_ref[...] = c2.astype(jnp.int32)[None]

    @pl.when(step == 0)
    def _init():
        psum_ref[...] = jnp.zeros_like(psum_ref)
        cnt_ref[...] = jnp.zeros_like(cnt_ref)

    psum_ref[...] += jnp.sum(probs, axis=0, keepdims=True)
    cnt_ref[...] += c1 + c2

    @pl.when(step == pl.num_programs(0) - 1)
    def _fin():
        frac_tok = cnt_ref[...] / float(N * TOPK)
        frac_prob = psum_ref[...] / float(N)
        aux_ref[0, 0] = float(E) * jnp.sum(frac_tok * frac_prob)


def _router(x, gate_weight, shared_gate_w):
    grid = (N // BMR,)
    return pl.pallas_call(
        _router_body,
        grid=grid,
        in_specs=[
            pl.BlockSpec((BMR, H), lambda i: (i, 0)),
            pl.BlockSpec((E, H), lambda i: (0, 0)),
            pl.BlockSpec((1, H), lambda i: (0, 0)),
        ],
        out_specs=[
            pl.BlockSpec((BMR, 1), lambda i: (i, 0)),
            pl.BlockSpec((BMR, 1), lambda i: (i, 0)),
            pl.BlockSpec((BMR, 1), lambda i: (i, 0)),
            pl.BlockSpec((BMR, 1), lambda i: (i, 0)),
            pl.BlockSpec(memory_space=pltpu.SMEM),
            pl.BlockSpec((1, 1, E), lambda i: (i, 0, 0)),
            pl.BlockSpec((1, 1, E), lambda i: (i, 0, 0)),
            pl.BlockSpec((BMR, 1), lambda i: (i, 0)),
        ],
        out_shape=[
            jax.ShapeDtypeStruct((N, 1), jnp.int32),
            jax.ShapeDtypeStruct((N, 1), jnp.int32),
            jax.ShapeDtypeStruct((N, 1), jnp.float32),
            jax.ShapeDtypeStruct((N, 1), jnp.float32),
            jax.ShapeDtypeStruct((1, 1), jnp.float32),
            jax.ShapeDtypeStruct((NCHUNK // 2, 1, E), jnp.int32),
            jax.ShapeDtypeStruct((NCHUNK // 2, 1, E), jnp.int32),
            jax.ShapeDtypeStruct((N, 1), jnp.float32),
        ],
        scratch_shapes=[
            pltpu.VMEM((1, E), jnp.float32),
            pltpu.VMEM((1, E), jnp.float32),
        ],
    )(x, gate_weight, shared_gate_w)


def _dispatch_body(eid_hbm, bases_hbm, x_hbm,
                   xs_hbm, pos_hbm,
                   eid_v, dest_v, bases_v, buf0, buf1, sem0, sem1):
    w = lax.axis_index("s") * 2 + lax.axis_index("c")
    base = pl.multiple_of(w * BMR, BMR)
    pltpu.sync_copy(eid_hbm.at[pl.ds(base, BMR)], eid_v)
    pltpu.sync_copy(bases_hbm.at[w], bases_v)

    lanes = lax.broadcasted_iota(jnp.int32, (16,), 0)
    dnums = lax.GatherDimensionNumbers(
        offset_dims=(), collapsed_slice_dims=(0,), start_index_map=(0,))

    def bcast_lane(vec, e):
        idx = jnp.full((16, 1), e, jnp.int32)
        return lax.gather(vec, idx, dnums, (1,),
                          mode=lax.GatherScatterMode.PROMISE_IN_BOUNDS)

    bv = bases_v[...]
    for i in range(BMR // 16):
        v = eid_v[pl.ds(i * 16, 16)]
        dest = jnp.zeros((16,), jnp.int32)
        for e in range(E):
            m = v == e
            csum = plsc.cumsum(jnp.where(m, 1, 0).astype(jnp.int32))
            dest = jnp.where(m, bcast_lane(bv, e) + csum - 1, dest)
            pc = plsc.all_reduce_population_count(m)
            bv = bv + jnp.where(lanes == e, pc, 0)
        dest_v[i // 2, pl.ds((i % 2) * 16, 16)] = dest

    pltpu.sync_copy(dest_v, pos_hbm.at[w])

    tstart = pl.multiple_of((w % (NCHUNK // 2)) * BMR, BMR)
    bufs = (buf0, buf1)
    sems = (sem0, sem1)
    descs = [None, None]
    for c in range(8):
        if descs[c % 2] is not None:
            descs[c % 2].wait()
        pltpu.sync_copy(x_hbm.at[pl.ds(tstart + c * 32, 32)], bufs[c % 2])
        descs[c % 2] = pltpu.async_copy(
            bufs[c % 2], xs_hbm.at[dest_v.at[c]], sems[c % 2])
    descs[0].wait()
    descs[1].wait()


def _dispatch(eid, bases, x):
    mesh = plsc.VectorSubcoreMesh(core_axis_name="c", subcore_axis_name="s")
    f = functools.partial(
        pl.kernel,
        mesh=mesh,
        compiler_params=pltpu.CompilerParams(needs_layout_passes=False),
        out_type=[
            jax.ShapeDtypeStruct((CAP, H), jnp.float32),
            jax.ShapeDtypeStruct((NCHUNK, 8, 32), jnp.int32),
        ],
        scratch_types=[
            pltpu.VMEM((BMR,), jnp.int32),
            pltpu.VMEM((8, 32), jnp.int32),
            pltpu.VMEM((16,), jnp.int32),
            pltpu.VMEM((32, H), jnp.float32),
            pltpu.VMEM((32, H), jnp.float32),
            pltpu.SemaphoreType.DMA,
            pltpu.SemaphoreType.DMA,
        ],
    )(_dispatch_body)
    return f(eid, bases, x)


def _ffn_body(be_ref, nb_ref, xs_ref, gu_ref, dn_ref, ys_ref):
    b = pl.program_id(0)

    @pl.when(b < nb_ref[0])
    def _():
        x = xs_ref[...].astype(jnp.bfloat16)
        gu = gu_ref[0].astype(jnp.bfloat16)
        gup = lax.dot_general(x, gu, (((1,), (1,)), ((), ())),
                              preferred_element_type=jnp.float32)
        g = gup[:, :I_MOE]
        u = gup[:, I_MOE:]
        h = (g * jax.nn.sigmoid(g) * u).astype(jnp.bfloat16)
        dn = dn_ref[0].astype(jnp.bfloat16)
        ys_ref[...] = lax.dot_general(h, dn, (((1,), (1,)), ((), ())),
                                      preferred_element_type=jnp.float32)


def _grouped_ffn(block_expert, nblocks, xs, gate_up_proj, down_proj):
    grid_spec = pltpu.PrefetchScalarGridSpec(
        num_scalar_prefetch=2,
        grid=(NB,),
        in_specs=[
            pl.BlockSpec((BM, H),
                         lambda b, be, nb: (jnp.minimum(b, nb[0] - 1), 0)),
            pl.BlockSpec((1, 2 * I_MOE, H), lambda b, be, nb: (be[b], 0, 0)),
            pl.BlockSpec((1, H, I_MOE), lambda b, be, nb: (be[b], 0, 0)),
        ],
        out_specs=pl.BlockSpec((BM, H),
                               lambda b, be, nb: (jnp.minimum(b, nb[0] - 1), 0)),
    )
    return pl.pallas_call(
        _ffn_body,
        grid_spec=grid_spec,
        out_shape=jax.ShapeDtypeStruct((CAP, H), jnp.float32),
    )(block_expert, nblocks, xs, gate_up_proj, down_proj)


def _shared_half_body(x_ref, g_ref, u_ref, d_ref, out_ref):
    x = x_ref[...]
    gw = g_ref[...].astype(jnp.bfloat16)
    uw = u_ref[...].astype(jnp.bfloat16)
    g = lax.dot_general(x, gw, (((1,), (1,)), ((), ())),
                        preferred_element_type=jnp.float32)
    u = lax.dot_general(x, uw, (((1,), (1,)), ((), ())),
                        preferred_element_type=jnp.float32)
    h = (g * jax.nn.sigmoid(g) * u).astype(jnp.bfloat16)
    dw = d_ref[...].astype(jnp.bfloat16)
    out_ref[...] = lax.dot_general(h, dw, (((1,), (1,)), ((), ())),
                                   preferred_element_type=jnp.float32)


def _shared_final_body(x_ref, g_ref, u_ref, d_ref, sh0_ref, z_ref, ymoe_ref,
                       prev_ref, out_ref):
    del prev_ref
    x = x_ref[...]
    gw = g_ref[...].astype(jnp.bfloat16)
    uw = u_ref[...].astype(jnp.bfloat16)
    g = lax.dot_general(x, gw, (((1,), (1,)), ((), ())),
                        preferred_element_type=jnp.float32)
    u = lax.dot_general(x, uw, (((1,), (1,)), ((), ())),
                        preferred_element_type=jnp.float32)
    h = (g * jax.nn.sigmoid(g) * u).astype(jnp.bfloat16)
    dw = d_ref[...].astype(jnp.bfloat16)
    contrib = lax.dot_general(h, dw, (((1,), (1,)), ((), ())),
                              preferred_element_type=jnp.float32)
    out_ref[...] = ((sh0_ref[...] + contrib) * jax.nn.sigmoid(z_ref[...])
                    + ymoe_ref[...])


NH = N
SBH = NH // BMS


def _shared_half(xb, sh_gate, sh_up, sh_down, half):
    return pl.pallas_call(
        _shared_half_body,
        grid=(SBH,),
        in_specs=[
            pl.BlockSpec((BMS, H), lambda m: (m + half * SBH, 0)),
            pl.BlockSpec((BIS, H), lambda m: (0, 0)),
            pl.BlockSpec((BIS, H), lambda m: (0, 0)),
            pl.BlockSpec((H, BIS), lambda m: (0, 0)),
        ],
        out_specs=pl.BlockSpec((BMS, H), lambda m: (m, 0)),
        out_shape=jax.ShapeDtypeStruct((NH, H), jnp.float32),
    )(xb, sh_gate, sh_up, sh_down)


def _shared_final(xb, sh_gate, sh_up, sh_down, sh0, z, ymoe, prev, half):
    in_specs = [
        pl.BlockSpec((BMS, H), lambda m: (m + half * SBH, 0)),
        pl.BlockSpec((BIS, H), lambda m: (1, 0)),
        pl.BlockSpec((BIS, H), lambda m: (1, 0)),
        pl.BlockSpec((H, BIS), lambda m: (0, 1)),
        pl.BlockSpec((BMS, H), lambda m: (m, 0)),
        pl.BlockSpec((BMS, 1), lambda m: (m + half * SBH, 0)),
        pl.BlockSpec((BMS, H), lambda m: (m, 0)),
    ]
    args = [xb, sh_gate, sh_up, sh_down, sh0, z, ymoe]
    aliases = {}
    body = _shared_final_body
    if prev is not None:
        in_specs.append(pl.BlockSpec(memory_space=pl.ANY))
        args.append(prev)
        aliases = {7: 0}
    else:
        def body(x_ref, g_ref, u_ref, d_ref, sh0_ref, z_ref, ymoe_ref,
                 out_ref):
            _shared_final_body(x_ref, g_ref, u_ref, d_ref, sh0_ref, z_ref,
                               ymoe_ref, None, out_ref)
    return pl.pallas_call(
        body,
        grid=(SBH,),
        in_specs=in_specs,
        out_specs=pl.BlockSpec((BMS, H), lambda m: (m + half * SBH, 0)),
        out_shape=jax.ShapeDtypeStruct((N, H), jnp.float32),
        input_output_aliases=aliases,
    )(*args)


def _combine_body(half, ys_hbm, pos_hbm, w1_hbm, w2_hbm, out_hbm,
                  idx1, idx2, wv1, wv2, y1, y2, acc, sem1, sem2):
    w = lax.axis_index("s") * 2 + lax.axis_index("c")
    tpt = NH // NTILE
    for sub in range(tpt // CSUB):
        lt0 = pl.multiple_of(w * tpt + sub * CSUB, CSUB)
        t0 = pl.multiple_of(half * NH + w * tpt + sub * CSUB, CSUB)
        pltpu.sync_copy(pos_hbm.at[pl.ds(t0, CSUB)], idx1)
        pltpu.sync_copy(pos_hbm.at[pl.ds(N + t0, CSUB)], idx2)
        d1 = pltpu.async_copy(ys_hbm.at[idx1], y1, sem1)
        d2 = pltpu.async_copy(ys_hbm.at[idx2], y2, sem2)
        pltpu.sync_copy(w1_hbm.at[pl.ds(t0, CSUB)], wv1)
        pltpu.sync_copy(w2_hbm.at[pl.ds(t0, CSUB)], wv2)
        d1.wait()
        d2.wait()

        def row(r, _):
            ri = jnp.full((16,), r, jnp.int32)
            b1 = plsc.load_gather(wv1, [ri])
            b2 = plsc.load_gather(wv2, [ri])
            for j in range(H // 16):
                sl = pl.ds(j * 16, 16)
                acc[r, sl] = b1 * y1[r, sl] + b2 * y2[r, sl]
            return 0

        lax.fori_loop(0, CSUB, row, 0)
        pltpu.sync_copy(acc, out_hbm.at[pl.ds(lt0, CSUB)])


def _combine(ys, pos_flat, w1, w2, half):
    mesh = plsc.VectorSubcoreMesh(core_axis_name="c", subcore_axis_name="s")
    f = functools.partial(
        pl.kernel,
        mesh=mesh,
        compiler_params=pltpu.CompilerParams(needs_layout_passes=False),
        out_type=jax.ShapeDtypeStruct((NH, H), jnp.float32),
        scratch_types=[
            pltpu.VMEM((CSUB,), jnp.int32),
            pltpu.VMEM((CSUB,), jnp.int32),
            pltpu.VMEM((CSUB,), jnp.float32),
            pltpu.VMEM((CSUB,), jnp.float32),
            pltpu.VMEM((CSUB, H), jnp.float32),
            pltpu.VMEM((CSUB, H), jnp.float32),
            pltpu.VMEM((CSUB, H), jnp.float32),
            pltpu.SemaphoreType.DMA,
            pltpu.SemaphoreType.DMA,
        ],
    )(functools.partial(_combine_body, half))
    return f(ys, pos_flat, w1, w2)


def kernel(hidden_states, gate_weight, gate_up_proj, down_proj,
           sh_gate_proj, sh_up_proj, sh_down_proj, shared_gate_w):
    x = hidden_states.reshape(N, H)

    a1, a2, w1, w2, aux, c1, c2, z = _router(x, gate_weight, shared_gate_w)

    cnts = jnp.concatenate([c1.reshape(NCHUNK // 2, E),
                            c2.reshape(NCHUNK // 2, E)], axis=0)
    counts = jnp.sum(cnts, axis=0)
    padded = ((counts + BM - 1) // BM) * BM
    poff = jnp.concatenate([jnp.zeros((1,), jnp.int32),
                            jnp.cumsum(padded)]).astype(jnp.int32)
    tilebase = poff[:E][None, :] + jnp.cumsum(cnts, axis=0) - cnts
    bases = jnp.pad(tilebase, ((0, 0), (0, 16 - E))).astype(jnp.int32)
    nblocks = (poff[E] // BM).reshape(1)
    bidx = jnp.arange(NB, dtype=jnp.int32) * BM
    block_expert = jnp.sum(
        (poff[1:E + 1][None, :] <= bidx[:, None]).astype(jnp.int32), axis=1)
    block_expert = jnp.minimum(block_expert, E - 1)

    eid = jnp.concatenate([a1[:, 0], a2[:, 0]])
    xb = x.astype(jnp.bfloat16)

    xs, pos = _dispatch(eid, bases, x)
    ys = _grouped_ffn(block_expert, nblocks, xs, gate_up_proj, down_proj)

    pos_flat = pos.reshape(P)
    w1f, w2f = w1.reshape(N), w2.reshape(N)
    sh0 = _shared_half(xb, sh_gate_proj, sh_up_proj, sh_down_proj, 0)
    ymoe = _combine(ys, pos_flat, w1f, w2f, 0)
    out = _shared_final(xb, sh_gate_proj, sh_up_proj, sh_down_proj,
                        sh0, z, ymoe, None, 0)
    return out.reshape(B, S, H), aux[0, 0]


# speedup vs baseline: 1.1771x; 1.0595x over previous
import functools

import jax
import jax.numpy as jnp
from jax import lax
from jax.experimental import pallas as pl
from jax.experimental.pallas import tpu as pltpu
from jax.experimental.pallas import tpu_sc as plsc

B, S, H = 2, 2048, 1024
E, TOPK = 8, 2
I_MOE = 1024
I_SH = 2816
N = B * S
P = N * TOPK

BM = 256
CAP = P + E * BM
NB = CAP // BM

BMR = 256
NCHUNK = P // BMR
BRT = 512
BMS = 512
BIS = 1408
NIS = I_SH // BIS

NTILE = 32
TPT = N // NTILE
CSUB = 32


def _router_body(x_ref, gw_ref, sgw_ref, a1_ref, a2_ref, w1_ref, w2_ref,
                 aux_ref, c1_ref, c2_ref, z_ref, xb_ref, psum_ref, cnt_ref):
    step = pl.program_id(0)
    x = x_ref[...]
    xb_ref[...] = x.astype(jnp.bfloat16)
    z_ref[...] = lax.dot_general(x, sgw_ref[...], (((1,), (1,)), ((), ())),
                                 preferred_element_type=jnp.float32)
    logits = lax.dot_general(x, gw_ref[...], (((1,), (1,)), ((), ())),
                             preferred_element_type=jnp.float32)
    ii = lax.broadcasted_iota(jnp.int32, logits.shape, 1)
    m1 = jnp.max(logits, axis=1, keepdims=True)
    a1 = jnp.min(jnp.where(logits >= m1, ii, E), axis=1, keepdims=True)
    l2 = jnp.where(ii == a1, -jnp.inf, logits)
    m2 = jnp.max(l2, axis=1, keepdims=True)
    a2 = jnp.min(jnp.where(l2 >= m2, ii, E), axis=1, keepdims=True)
    ex = jnp.exp(logits - m1)
    s = jnp.sum(ex, axis=1, keepdims=True)
    a1_ref[...] = a1
    a2_ref[...] = a2
    w1_ref[...] = 1.0 / s
    w2_ref[...] = jnp.exp(m2 - m1) / s

    probs = ex / s
    oh1 = (ii == a1).astype(jnp.float32)
    oh2 = (ii == a2).astype(jnp.float32)
    c1 = jnp.sum(oh1, axis=0, keepdims=True)
    c2 = jnp.sum(oh2, axis=0, keepdims=True)
    c1a = jnp.sum(oh1[:BMR], axis=0, keepdims=True)
    c2a = jnp.sum(oh2[:BMR], axis=0, keepdims=True)
    c1_ref[...] = jnp.concatenate(
        [c1a[None], (c1 - c1a)[None]], axis=0).astype(jnp.int32)
    c2_ref[...] = jnp.concatenate(
        [c2a[None], (c2 - c2a)[None]], axis=0).astype(jnp.int32)

    @pl.when(step == 0)
    def _init():
        psum_ref[...] = jnp.zeros_like(psum_ref)
        cnt_ref[...] = jnp.zeros_like(cnt_ref)

    psum_ref[...] += jnp.sum(probs, axis=0, keepdims=True)
    cnt_ref[...] += c1 + c2

    @pl.when(step == pl.num_programs(0) - 1)
    def _fin():
        frac_tok = cnt_ref[...] / float(N * TOPK)
        frac_prob = psum_ref[...] / float(N)
        aux_ref[0, 0] = float(E) * jnp.sum(frac_tok * frac_prob)


def _router(x, gate_weight, shared_gate_w):
    grid = (N // BRT,)
    return pl.pallas_call(
        _router_body,
        grid=grid,
        in_specs=[
            pl.BlockSpec((BRT, H), lambda i: (i, 0)),
            pl.BlockSpec((E, H), lambda i: (0, 0)),
            pl.BlockSpec((1, H), lambda i: (0, 0)),
        ],
        out_specs=[
            pl.BlockSpec((BRT, 1), lambda i: (i, 0)),
            pl.BlockSpec((BRT, 1), lambda i: (i, 0)),
            pl.BlockSpec((BRT, 1), lambda i: (i, 0)),
            pl.BlockSpec((BRT, 1), lambda i: (i, 0)),
            pl.BlockSpec(memory_space=pltpu.SMEM),
            pl.BlockSpec((2, 1, E), lambda i: (i, 0, 0)),
            pl.BlockSpec((2, 1, E), lambda i: (i, 0, 0)),
            pl.BlockSpec((BRT, 1), lambda i: (i, 0)),
            pl.BlockSpec((BRT, H), lambda i: (i, 0)),
        ],
        out_shape=[
            jax.ShapeDtypeStruct((N, 1), jnp.int32),
            jax.ShapeDtypeStruct((N, 1), jnp.int32),
            jax.ShapeDtypeStruct((N, 1), jnp.float32),
            jax.ShapeDtypeStruct((N, 1), jnp.float32),
            jax.ShapeDtypeStruct((1, 1), jnp.float32),
            jax.ShapeDtypeStruct((NCHUNK // 2, 1, E), jnp.int32),
            jax.ShapeDtypeStruct((NCHUNK // 2, 1, E), jnp.int32),
            jax.ShapeDtypeStruct((N, 1), jnp.float32),
            jax.ShapeDtypeStruct((N, H), jnp.bfloat16),
        ],
        scratch_shapes=[
            pltpu.VMEM((1, E), jnp.float32),
            pltpu.VMEM((1, E), jnp.float32),
        ],
    )(x, gate_weight, shared_gate_w)


def _dispatch_body(eid_hbm, bases_hbm, x_hbm,
                   xs_hbm, pos_hbm,
                   eid_v, dest_v, bases_v, buf0, buf1, sem0, sem1):
    w = lax.axis_index("s") * 2 + lax.axis_index("c")
    base = pl.multiple_of(w * BMR, BMR)
    pltpu.sync_copy(eid_hbm.at[pl.ds(base, BMR)], eid_v)
    pltpu.sync_copy(bases_hbm.at[w], bases_v)

    lanes = lax.broadcasted_iota(jnp.int32, (16,), 0)
    dnums = lax.GatherDimensionNumbers(
        offset_dims=(), collapsed_slice_dims=(0,), start_index_map=(0,))

    def bcast_lane(vec, e):
        idx = jnp.full((16, 1), e, jnp.int32)
        return lax.gather(vec, idx, dnums, (1,),
                          mode=lax.GatherScatterMode.PROMISE_IN_BOUNDS)

    bv = bases_v[...]
    for i in range(BMR // 16):
        v = eid_v[pl.ds(i * 16, 16)]
        dest = jnp.zeros((16,), jnp.int32)
        for e in range(E):
            m = v == e
            csum = plsc.cumsum(jnp.where(m, 1, 0).astype(jnp.int32))
            dest = jnp.where(m, bcast_lane(bv, e) + csum - 1, dest)
            pc = plsc.all_reduce_population_count(m)
            bv = bv + jnp.where(lanes == e, pc, 0)
        dest_v[i // 2, pl.ds((i % 2) * 16, 16)] = dest

    pltpu.sync_copy(dest_v, pos_hbm.at[w])

    tstart = pl.multiple_of((w % (NCHUNK // 2)) * BMR, BMR)
    bufs = (buf0, buf1)
    sems = (sem0, sem1)
    descs = [None, None]
    for c in range(8):
        if descs[c % 2] is not None:
            descs[c % 2].wait()
        pltpu.sync_copy(x_hbm.at[pl.ds(tstart + c * 32, 32)], bufs[c % 2])
        descs[c % 2] = pltpu.async_copy(
            bufs[c % 2], xs_hbm.at[dest_v.at[c]], sems[c % 2])
    descs[0].wait()
    descs[1].wait()


def _dispatch(eid, bases, x):
    mesh = plsc.VectorSubcoreMesh(core_axis_name="c", subcore_axis_name="s")
    f = functools.partial(
        pl.kernel,
        mesh=mesh,
        compiler_params=pltpu.CompilerParams(needs_layout_passes=False),
        out_type=[
            jax.ShapeDtypeStruct((CAP, H), jnp.float32),
            jax.ShapeDtypeStruct((NCHUNK, 8, 32), jnp.int32),
        ],
        scratch_types=[
            pltpu.VMEM((BMR,), jnp.int32),
            pltpu.VMEM((8, 32), jnp.int32),
            pltpu.VMEM((16,), jnp.int32),
            pltpu.VMEM((32, H), jnp.float32),
            pltpu.VMEM((32, H), jnp.float32),
            pltpu.SemaphoreType.DMA,
            pltpu.SemaphoreType.DMA,
        ],
    )(_dispatch_body)
    return f(eid, bases, x)


def _ffn_body(be_ref, nb_ref, xs_ref, gu_ref, dn_ref, ys_ref):
    b = pl.program_id(0)

    @pl.when(b < nb_ref[0])
    def _():
        x = xs_ref[...].astype(jnp.bfloat16)
        gu = gu_ref[0].astype(jnp.bfloat16)
        gup = lax.dot_general(x, gu, (((1,), (1,)), ((), ())),
                              preferred_element_type=jnp.float32)
        g = gup[:, :I_MOE]
        u = gup[:, I_MOE:]
        h = (g * jax.nn.sigmoid(g) * u).astype(jnp.bfloat16)
        dn = dn_ref[0].astype(jnp.bfloat16)
        ys_ref[...] = lax.dot_general(h, dn, (((1,), (1,)), ((), ())),
                                      preferred_element_type=jnp.float32)


def _grouped_ffn(block_expert, nblocks, xs, gate_up_proj, down_proj):
    grid_spec = pltpu.PrefetchScalarGridSpec(
        num_scalar_prefetch=2,
        grid=(NB,),
        in_specs=[
            pl.BlockSpec((BM, H),
                         lambda b, be, nb: (jnp.minimum(b, nb[0] - 1), 0)),
            pl.BlockSpec((1, 2 * I_MOE, H), lambda b, be, nb: (be[b], 0, 0)),
            pl.BlockSpec((1, H, I_MOE), lambda b, be, nb: (be[b], 0, 0)),
        ],
        out_specs=pl.BlockSpec((BM, H),
                               lambda b, be, nb: (jnp.minimum(b, nb[0] - 1), 0)),
    )
    return pl.pallas_call(
        _ffn_body,
        grid_spec=grid_spec,
        out_shape=jax.ShapeDtypeStruct((CAP, H), jnp.float32),
    )(block_expert, nblocks, xs, gate_up_proj, down_proj)


def _shared_half_body(x_ref, g_ref, u_ref, d_ref, out_ref):
    x = x_ref[...]
    gw = g_ref[...].astype(jnp.bfloat16)
    uw = u_ref[...].astype(jnp.bfloat16)
    g = lax.dot_general(x, gw, (((1,), (1,)), ((), ())),
                        preferred_element_type=jnp.float32)
    u = lax.dot_general(x, uw, (((1,), (1,)), ((), ())),
                        preferred_element_type=jnp.float32)
    h = (g * jax.nn.sigmoid(g) * u).astype(jnp.bfloat16)
    dw = d_ref[...].astype(jnp.bfloat16)
    out_ref[...] = lax.dot_general(h, dw, (((1,), (1,)), ((), ())),
                                   preferred_element_type=jnp.float32)


def _shared_final_body(x_ref, g_ref, u_ref, d_ref, sh0_ref, z_ref, ymoe_ref,
                       prev_ref, out_ref):
    del prev_ref
    x = x_ref[...]
    gw = g_ref[...].astype(jnp.bfloat16)
    uw = u_ref[...].astype(jnp.bfloat16)
    g = lax.dot_general(x, gw, (((1,), (1,)), ((), ())),
                        preferred_element_type=jnp.float32)
    u = lax.dot_general(x, uw, (((1,), (1,)), ((), ())),
                        preferred_element_type=jnp.float32)
    h = (g * jax.nn.sigmoid(g) * u).astype(jnp.bfloat16)
    dw = d_ref[...].astype(jnp.bfloat16)
    contrib = lax.dot_general(h, dw, (((1,), (1,)), ((), ())),
                              preferred_element_type=jnp.float32)
    out_ref[...] = ((sh0_ref[...] + contrib) * jax.nn.sigmoid(z_ref[...])
                    + ymoe_ref[...])


NH = N
SBH = NH // BMS


def _shared_half(xb, sh_gate, sh_up, sh_down, half):
    return pl.pallas_call(
        _shared_half_body,
        grid=(SBH,),
        in_specs=[
            pl.BlockSpec((BMS, H), lambda m: (m + half * SBH, 0)),
            pl.BlockSpec((BIS, H), lambda m: (0, 0)),
            pl.BlockSpec((BIS, H), lambda m: (0, 0)),
            pl.BlockSpec((H, BIS), lambda m: (0, 0)),
        ],
        out_specs=pl.BlockSpec((BMS, H), lambda m: (m, 0)),
        out_shape=jax.ShapeDtypeStruct((NH, H), jnp.float32),
    )(xb, sh_gate, sh_up, sh_down)


def _shared_final(xb, sh_gate, sh_up, sh_down, sh0, z, ymoe, prev, half):
    in_specs = [
        pl.BlockSpec((BMS, H), lambda m: (m + half * SBH, 0)),
        pl.BlockSpec((BIS, H), lambda m: (1, 0)),
        pl.BlockSpec((BIS, H), lambda m: (1, 0)),
        pl.BlockSpec((H, BIS), lambda m: (0, 1)),
        pl.BlockSpec((BMS, H), lambda m: (m, 0)),
        pl.BlockSpec((BMS, 1), lambda m: (m + half * SBH, 0)),
        pl.BlockSpec((BMS, H), lambda m: (m, 0)),
    ]
    args = [xb, sh_gate, sh_up, sh_down, sh0, z, ymoe]
    aliases = {}
    body = _shared_final_body
    if prev is not None:
        in_specs.append(pl.BlockSpec(memory_space=pl.ANY))
        args.append(prev)
        aliases = {7: 0}
    else:
        def body(x_ref, g_ref, u_ref, d_ref, sh0_ref, z_ref, ymoe_ref,
                 out_ref):
            _shared_final_body(x_ref, g_ref, u_ref, d_ref, sh0_ref, z_ref,
                               ymoe_ref, None, out_ref)
    return pl.pallas_call(
        body,
        grid=(SBH,),
        in_specs=in_specs,
        out_specs=pl.BlockSpec((BMS, H), lambda m: (m + half * SBH, 0)),
        out_shape=jax.ShapeDtypeStruct((N, H), jnp.float32),
        input_output_aliases=aliases,
    )(*args)


def _combine_body(half, ys_hbm, pos_hbm, w1_hbm, w2_hbm, out_hbm,
                  idx1, idx2, wv1, wv2, y1, y2, acc, sem1, sem2):
    w = lax.axis_index("s") * 2 + lax.axis_index("c")
    tpt = NH // NTILE
    for sub in range(tpt // CSUB):
        lt0 = pl.multiple_of(w * tpt + sub * CSUB, CSUB)
        t0 = pl.multiple_of(half * NH + w * tpt + sub * CSUB, CSUB)
        pltpu.sync_copy(pos_hbm.at[pl.ds(t0, CSUB)], idx1)
        pltpu.sync_copy(pos_hbm.at[pl.ds(N + t0, CSUB)], idx2)
        d1 = pltpu.async_copy(ys_hbm.at[idx1], y1, sem1)
        d2 = pltpu.async_copy(ys_hbm.at[idx2], y2, sem2)
        pltpu.sync_copy(w1_hbm.at[pl.ds(t0, CSUB)], wv1)
        pltpu.sync_copy(w2_hbm.at[pl.ds(t0, CSUB)], wv2)
        d1.wait()
        d2.wait()

        def row(r, _):
            ri = jnp.full((16,), r, jnp.int32)
            b1 = plsc.load_gather(wv1, [ri])
            b2 = plsc.load_gather(wv2, [ri])
            for j in range(H // 16):
                sl = pl.ds(j * 16, 16)
                acc[r, sl] = b1 * y1[r, sl] + b2 * y2[r, sl]
            return 0

        lax.fori_loop(0, CSUB, row, 0)
        pltpu.sync_copy(acc, out_hbm.at[pl.ds(lt0, CSUB)])


def _combine(ys, pos_flat, w1, w2, half):
    mesh = plsc.VectorSubcoreMesh(core_axis_name="c", subcore_axis_name="s")
    f = functools.partial(
        pl.kernel,
        mesh=mesh,
        compiler_params=pltpu.CompilerParams(needs_layout_passes=False),
        out_type=jax.ShapeDtypeStruct((NH, H), jnp.float32),
        scratch_types=[
            pltpu.VMEM((CSUB,), jnp.int32),
            pltpu.VMEM((CSUB,), jnp.int32),
            pltpu.VMEM((CSUB,), jnp.float32),
            pltpu.VMEM((CSUB,), jnp.float32),
            pltpu.VMEM((CSUB, H), jnp.float32),
            pltpu.VMEM((CSUB, H), jnp.float32),
            pltpu.VMEM((CSUB, H), jnp.float32),
            pltpu.SemaphoreType.DMA,
            pltpu.SemaphoreType.DMA,
        ],
    )(functools.partial(_combine_body, half))
    return f(ys, pos_flat, w1, w2)


def kernel(hidden_states, gate_weight, gate_up_proj, down_proj,
           sh_gate_proj, sh_up_proj, sh_down_proj, shared_gate_w):
    x = hidden_states.reshape(N, H)

    a1, a2, w1, w2, aux, c1, c2, z, xb = _router(x, gate_weight,
                                                 shared_gate_w)

    cnts = jnp.concatenate([c1.reshape(NCHUNK // 2, E),
                            c2.reshape(NCHUNK // 2, E)], axis=0)
    counts = jnp.sum(cnts, axis=0)
    padded = ((counts + BM - 1) // BM) * BM
    poff = jnp.concatenate([jnp.zeros((1,), jnp.int32),
                            jnp.cumsum(padded)]).astype(jnp.int32)
    tilebase = poff[:E][None, :] + jnp.cumsum(cnts, axis=0) - cnts
    bases = jnp.pad(tilebase, ((0, 0), (0, 16 - E))).astype(jnp.int32)
    nblocks = (poff[E] // BM).reshape(1)
    bidx = jnp.arange(NB, dtype=jnp.int32) * BM
    block_expert = jnp.sum(
        (poff[1:E + 1][None, :] <= bidx[:, None]).astype(jnp.int32), axis=1)
    block_expert = jnp.minimum(block_expert, E - 1)

    eid = jnp.concatenate([a1[:, 0], a2[:, 0]])

    xs, pos = _dispatch(eid, bases, x)
    ys = _grouped_ffn(block_expert, nblocks, xs, gate_up_proj, down_proj)

    pos_flat = pos.reshape(P)
    w1f, w2f = w1.reshape(N), w2.reshape(N)
    sh0 = _shared_half(xb, sh_gate_proj, sh_up_proj, sh_down_proj, 0)
    ymoe = _combine(ys, pos_flat, w1f, w2f, 0)
    out = _shared_final(xb, sh_gate_proj, sh_up_proj, sh_down_proj,
                        sh0, z, ymoe, None, 0)
    return out.reshape(B, S, H), aux[0, 0]


# speedup vs baseline: 1.1778x; 1.0006x over previous
import functools

import jax
import jax.numpy as jnp
from jax import lax
from jax.experimental import pallas as pl
from jax.experimental.pallas import tpu as pltpu
from jax.experimental.pallas import tpu_sc as plsc

B, S, H = 2, 2048, 1024
E, TOPK = 8, 2
I_MOE = 1024
I_SH = 2816
N = B * S
P = N * TOPK

BM = 256
CAP = P + E * BM
NB = CAP // BM

BMR = 256
NCHUNK = P // BMR
BRT = 512
BMS = 512
BIS = 1408
NIS = I_SH // BIS

NTILE = 32
TPT = N // NTILE
CSUB = 32


def _router_body(x_ref, gw_ref, sgw_ref, a1_ref, a2_ref, w1_ref, w2_ref,
                 aux_ref, c1_ref, c2_ref, z_ref, xb_ref, psum_ref, cnt_ref):
    step = pl.program_id(0)
    x = x_ref[...]
    xb_ref[...] = x.astype(jnp.bfloat16)
    z_ref[...] = lax.dot_general(x, sgw_ref[...], (((1,), (1,)), ((), ())),
                                 preferred_element_type=jnp.float32)
    logits = lax.dot_general(x, gw_ref[...], (((1,), (1,)), ((), ())),
                             preferred_element_type=jnp.float32)
    ii = lax.broadcasted_iota(jnp.int32, logits.shape, 1)
    m1 = jnp.max(logits, axis=1, keepdims=True)
    a1 = jnp.min(jnp.where(logits >= m1, ii, E), axis=1, keepdims=True)
    l2 = jnp.where(ii == a1, -jnp.inf, logits)
    m2 = jnp.max(l2, axis=1, keepdims=True)
    a2 = jnp.min(jnp.where(l2 >= m2, ii, E), axis=1, keepdims=True)
    ex = jnp.exp(logits - m1)
    s = jnp.sum(ex, axis=1, keepdims=True)
    a1_ref[...] = a1
    a2_ref[...] = a2
    w1_ref[...] = 1.0 / s
    w2_ref[...] = jnp.exp(m2 - m1) / s

    probs = ex / s
    oh1 = (ii == a1).astype(jnp.float32)
    oh2 = (ii == a2).astype(jnp.float32)
    c1 = jnp.sum(oh1, axis=0, keepdims=True)
    c2 = jnp.sum(oh2, axis=0, keepdims=True)
    c1a = jnp.sum(oh1[:BMR], axis=0, keepdims=True)
    c2a = jnp.sum(oh2[:BMR], axis=0, keepdims=True)
    c1_ref[...] = jnp.concatenate(
        [c1a[None], (c1 - c1a)[None]], axis=0).astype(jnp.int32)
    c2_ref[...] = jnp.concatenate(
        [c2a[None], (c2 - c2a)[None]], axis=0).astype(jnp.int32)

    @pl.when(step == 0)
    def _init():
        psum_ref[...] = jnp.zeros_like(psum_ref)
        cnt_ref[...] = jnp.zeros_like(cnt_ref)

    psum_ref[...] += jnp.sum(probs, axis=0, keepdims=True)
    cnt_ref[...] += c1 + c2

    @pl.when(step == pl.num_programs(0) - 1)
    def _fin():
        frac_tok = cnt_ref[...] / float(N * TOPK)
        frac_prob = psum_ref[...] / float(N)
        aux_ref[0, 0] = float(E) * jnp.sum(frac_tok * frac_prob)


def _router(x, gate_weight, shared_gate_w):
    grid = (N // BRT,)
    return pl.pallas_call(
        _router_body,
        grid=grid,
        in_specs=[
            pl.BlockSpec((BRT, H), lambda i: (i, 0)),
            pl.BlockSpec((E, H), lambda i: (0, 0)),
            pl.BlockSpec((1, H), lambda i: (0, 0)),
        ],
        out_specs=[
            pl.BlockSpec((BRT, 1), lambda i: (i, 0)),
            pl.BlockSpec((BRT, 1), lambda i: (i, 0)),
            pl.BlockSpec((BRT, 1), lambda i: (i, 0)),
            pl.BlockSpec((BRT, 1), lambda i: (i, 0)),
            pl.BlockSpec(memory_space=pltpu.SMEM),
            pl.BlockSpec((2, 1, E), lambda i: (i, 0, 0)),
            pl.BlockSpec((2, 1, E), lambda i: (i, 0, 0)),
            pl.BlockSpec((BRT, 1), lambda i: (i, 0)),
            pl.BlockSpec((BRT, H), lambda i: (i, 0)),
        ],
        out_shape=[
            jax.ShapeDtypeStruct((N, 1), jnp.int32),
            jax.ShapeDtypeStruct((N, 1), jnp.int32),
            jax.ShapeDtypeStruct((N, 1), jnp.float32),
            jax.ShapeDtypeStruct((N, 1), jnp.float32),
            jax.ShapeDtypeStruct((1, 1), jnp.float32),
            jax.ShapeDtypeStruct((NCHUNK // 2, 1, E), jnp.int32),
            jax.ShapeDtypeStruct((NCHUNK // 2, 1, E), jnp.int32),
            jax.ShapeDtypeStruct((N, 1), jnp.float32),
            jax.ShapeDtypeStruct((N, H), jnp.bfloat16),
        ],
        scratch_shapes=[
            pltpu.VMEM((1, E), jnp.float32),
            pltpu.VMEM((1, E), jnp.float32),
        ],
    )(x, gate_weight, shared_gate_w)


def _dispatch_body(eid_hbm, bases_hbm, x_hbm,
                   xs_hbm, pos_hbm,
                   eid_v, dest_v, bases_v, buf0, buf1, sem0, sem1):
    w = lax.axis_index("s") * 2 + lax.axis_index("c")
    base = pl.multiple_of(w * BMR, BMR)
    pltpu.sync_copy(eid_hbm.at[pl.ds(base, BMR)], eid_v)
    pltpu.sync_copy(bases_hbm.at[w], bases_v)

    lanes = lax.broadcasted_iota(jnp.int32, (16,), 0)
    dnums = lax.GatherDimensionNumbers(
        offset_dims=(), collapsed_slice_dims=(0,), start_index_map=(0,))

    def bcast_lane(vec, e):
        idx = jnp.full((16, 1), e, jnp.int32)
        return lax.gather(vec, idx, dnums, (1,),
                          mode=lax.GatherScatterMode.PROMISE_IN_BOUNDS)

    bv = bases_v[...]
    for i in range(BMR // 16):
        v = eid_v[pl.ds(i * 16, 16)]
        dest = jnp.zeros((16,), jnp.int32)
        for e in range(E):
            m = v == e
            csum = plsc.cumsum(jnp.where(m, 1, 0).astype(jnp.int32))
            dest = jnp.where(m, bcast_lane(bv, e) + csum - 1, dest)
            pc = plsc.all_reduce_population_count(m)
            bv = bv + jnp.where(lanes == e, pc, 0)
        dest_v[i // 2, pl.ds((i % 2) * 16, 16)] = dest

    pltpu.sync_copy(dest_v, pos_hbm.at[w])

    tstart = pl.multiple_of((w % (NCHUNK // 2)) * BMR, BMR)
    bufs = (buf0, buf1)
    sems = (sem0, sem1)
    descs = [None, None]
    for c in range(8):
        if descs[c % 2] is not None:
            descs[c % 2].wait()
        pltpu.sync_copy(x_hbm.at[pl.ds(tstart + c * 32, 32)], bufs[c % 2])
        descs[c % 2] = pltpu.async_copy(
            bufs[c % 2], xs_hbm.at[dest_v.at[c]], sems[c % 2])
    descs[0].wait()
    descs[1].wait()


def _dispatch(eid, bases, x):
    mesh = plsc.VectorSubcoreMesh(core_axis_name="c", subcore_axis_name="s")
    f = functools.partial(
        pl.kernel,
        mesh=mesh,
        compiler_params=pltpu.CompilerParams(needs_layout_passes=False),
        out_type=[
            jax.ShapeDtypeStruct((CAP, H), jnp.float32),
            jax.ShapeDtypeStruct((NCHUNK, 8, 32), jnp.int32),
        ],
        scratch_types=[
            pltpu.VMEM((BMR,), jnp.int32),
            pltpu.VMEM((8, 32), jnp.int32),
            pltpu.VMEM((16,), jnp.int32),
            pltpu.VMEM((32, H), jnp.float32),
            pltpu.VMEM((32, H), jnp.float32),
            pltpu.SemaphoreType.DMA,
            pltpu.SemaphoreType.DMA,
        ],
    )(_dispatch_body)
    return f(eid, bases, x)


def _ffn_body(be_ref, nb_ref, xs_ref, gu_ref, dn_ref, ys_ref):
    b = pl.program_id(0)

    @pl.when(b < nb_ref[0])
    def _():
        x = xs_ref[...].astype(jnp.bfloat16)
        gu = gu_ref[0].astype(jnp.bfloat16)
        gup = lax.dot_general(x, gu, (((1,), (1,)), ((), ())),
                              preferred_element_type=jnp.float32)
        g = gup[:, :I_MOE]
        u = gup[:, I_MOE:]
        h = (g * jax.nn.sigmoid(g) * u).astype(jnp.bfloat16)
        dn = dn_ref[0].astype(jnp.bfloat16)
        ys_ref[...] = lax.dot_general(h, dn, (((1,), (1,)), ((), ())),
                                      preferred_element_type=jnp.float32)


def _grouped_ffn(block_expert, nblocks, xs, gate_up_proj, down_proj):
    grid_spec = pltpu.PrefetchScalarGridSpec(
        num_scalar_prefetch=2,
        grid=(NB,),
        in_specs=[
            pl.BlockSpec((BM, H),
                         lambda b, be, nb: (jnp.minimum(b, nb[0] - 1), 0)),
            pl.BlockSpec((1, 2 * I_MOE, H), lambda b, be, nb: (be[b], 0, 0)),
            pl.BlockSpec((1, H, I_MOE), lambda b, be, nb: (be[b], 0, 0)),
        ],
        out_specs=pl.BlockSpec((BM, H),
                               lambda b, be, nb: (jnp.minimum(b, nb[0] - 1), 0)),
    )
    return pl.pallas_call(
        _ffn_body,
        grid_spec=grid_spec,
        out_shape=jax.ShapeDtypeStruct((CAP, H), jnp.float32),
    )(block_expert, nblocks, xs, gate_up_proj, down_proj)


def _shared_half_body(x_ref, g_ref, u_ref, d_ref, out_ref):
    x = x_ref[...]
    gw = g_ref[...].astype(jnp.bfloat16)
    uw = u_ref[...].astype(jnp.bfloat16)
    g = lax.dot_general(x, gw, (((1,), (1,)), ((), ())),
                        preferred_element_type=jnp.float32)
    u = lax.dot_general(x, uw, (((1,), (1,)), ((), ())),
                        preferred_element_type=jnp.float32)
    h = (g * jax.nn.sigmoid(g) * u).astype(jnp.bfloat16)
    dw = d_ref[...].astype(jnp.bfloat16)
    out_ref[...] = lax.dot_general(
        h, dw, (((1,), (1,)), ((), ())),
        preferred_element_type=jnp.float32).astype(jnp.bfloat16)


def _shared_final_body(x_ref, g_ref, u_ref, d_ref, sh0_ref, z_ref, ymoe_ref,
                       prev_ref, out_ref):
    del prev_ref
    x = x_ref[...]
    gw = g_ref[...].astype(jnp.bfloat16)
    uw = u_ref[...].astype(jnp.bfloat16)
    g = lax.dot_general(x, gw, (((1,), (1,)), ((), ())),
                        preferred_element_type=jnp.float32)
    u = lax.dot_general(x, uw, (((1,), (1,)), ((), ())),
                        preferred_element_type=jnp.float32)
    h = (g * jax.nn.sigmoid(g) * u).astype(jnp.bfloat16)
    dw = d_ref[...].astype(jnp.bfloat16)
    contrib = lax.dot_general(h, dw, (((1,), (1,)), ((), ())),
                              preferred_element_type=jnp.float32)
    out_ref[...] = ((sh0_ref[...].astype(jnp.float32) + contrib)
                    * jax.nn.sigmoid(z_ref[...])
                    + ymoe_ref[...].astype(jnp.float32))


NH = N
SBH = NH // BMS


def _shared_half(xb, sh_gate, sh_up, sh_down, half):
    return pl.pallas_call(
        _shared_half_body,
        grid=(SBH,),
        in_specs=[
            pl.BlockSpec((BMS, H), lambda m: (m + half * SBH, 0)),
            pl.BlockSpec((BIS, H), lambda m: (0, 0)),
            pl.BlockSpec((BIS, H), lambda m: (0, 0)),
            pl.BlockSpec((H, BIS), lambda m: (0, 0)),
        ],
        out_specs=pl.BlockSpec((BMS, H), lambda m: (m, 0)),
        out_shape=jax.ShapeDtypeStruct((NH, H), jnp.bfloat16),
    )(xb, sh_gate, sh_up, sh_down)


def _shared_final(xb, sh_gate, sh_up, sh_down, sh0, z, ymoe, prev, half):
    in_specs = [
        pl.BlockSpec((BMS, H), lambda m: (m + half * SBH, 0)),
        pl.BlockSpec((BIS, H), lambda m: (1, 0)),
        pl.BlockSpec((BIS, H), lambda m: (1, 0)),
        pl.BlockSpec((H, BIS), lambda m: (0, 1)),
        pl.BlockSpec((BMS, H), lambda m: (m, 0)),
        pl.BlockSpec((BMS, 1), lambda m: (m + half * SBH, 0)),
        pl.BlockSpec((BMS, H), lambda m: (m, 0)),
    ]
    args = [xb, sh_gate, sh_up, sh_down, sh0, z, ymoe]
    aliases = {}
    body = _shared_final_body
    if prev is not None:
        in_specs.append(pl.BlockSpec(memory_space=pl.ANY))
        args.append(prev)
        aliases = {7: 0}
    else:
        def body(x_ref, g_ref, u_ref, d_ref, sh0_ref, z_ref, ymoe_ref,
                 out_ref):
            _shared_final_body(x_ref, g_ref, u_ref, d_ref, sh0_ref, z_ref,
                               ymoe_ref, None, out_ref)
    return pl.pallas_call(
        body,
        grid=(SBH,),
        in_specs=in_specs,
        out_specs=pl.BlockSpec((BMS, H), lambda m: (m + half * SBH, 0)),
        out_shape=jax.ShapeDtypeStruct((N, H), jnp.float32),
        input_output_aliases=aliases,
    )(*args)


def _combine_body(half, ys_hbm, pos_hbm, w1_hbm, w2_hbm, out_hbm,
                  idx1, idx2, wv1, wv2, y1, y2, acc, sem1, sem2):
    w = lax.axis_index("s") * 2 + lax.axis_index("c")
    tpt = NH // NTILE
    for sub in range(tpt // CSUB):
        lt0 = pl.multiple_of(w * tpt + sub * CSUB, CSUB)
        t0 = pl.multiple_of(half * NH + w * tpt + sub * CSUB, CSUB)
        pltpu.sync_copy(pos_hbm.at[pl.ds(t0, CSUB)], idx1)
        pltpu.sync_copy(pos_hbm.at[pl.ds(N + t0, CSUB)], idx2)
        d1 = pltpu.async_copy(ys_hbm.at[idx1], y1, sem1)
        d2 = pltpu.async_copy(ys_hbm.at[idx2], y2, sem2)
        pltpu.sync_copy(w1_hbm.at[pl.ds(t0, CSUB)], wv1)
        pltpu.sync_copy(w2_hbm.at[pl.ds(t0, CSUB)], wv2)
        d1.wait()
        d2.wait()

        def row(r, _):
            ri = jnp.full((16,), r, jnp.int32)
            b1 = plsc.load_gather(wv1, [ri])
            b2 = plsc.load_gather(wv2, [ri])
            for j in range(H // 16):
                sl = pl.ds(j * 16, 16)
                acc[r, sl] = b1 * y1[r, sl] + b2 * y2[r, sl]
            return 0

        lax.fori_loop(0, CSUB, row, 0)
        pltpu.sync_copy(acc, out_hbm.at[pl.ds(lt0, CSUB)])


def _combine(ys, pos_flat, w1, w2, half):
    mesh = plsc.VectorSubcoreMesh(core_axis_name="c", subcore_axis_name="s")
    f = functools.partial(
        pl.kernel,
        mesh=mesh,
        compiler_params=pltpu.CompilerParams(needs_layout_passes=False),
        out_type=jax.ShapeDtypeStruct((NH, H), jnp.float32),
        scratch_types=[
            pltpu.VMEM((CSUB,), jnp.int32),
            pltpu.VMEM((CSUB,), jnp.int32),
            pltpu.VMEM((CSUB,), jnp.float32),
            pltpu.VMEM((CSUB,), jnp.float32),
            pltpu.VMEM((CSUB, H), jnp.float32),
            pltpu.VMEM((CSUB, H), jnp.float32),
            pltpu.VMEM((CSUB, H), jnp.float32),
            pltpu.SemaphoreType.DMA,
            pltpu.SemaphoreType.DMA,
        ],
    )(functools.partial(_combine_body, half))
    return f(ys, pos_flat, w1, w2)


def kernel(hidden_states, gate_weight, gate_up_proj, down_proj,
           sh_gate_proj, sh_up_proj, sh_down_proj, shared_gate_w):
    x = hidden_states.reshape(N, H)

    a1, a2, w1, w2, aux, c1, c2, z, xb = _router(x, gate_weight,
                                                 shared_gate_w)

    cnts = jnp.concatenate([c1.reshape(NCHUNK // 2, E),
                            c2.reshape(NCHUNK // 2, E)], axis=0)
    counts = jnp.sum(cnts, axis=0)
    padded = ((counts + BM - 1) // BM) * BM
    poff = jnp.concatenate([jnp.zeros((1,), jnp.int32),
                            jnp.cumsum(padded)]).astype(jnp.int32)
    tilebase = poff[:E][None, :] + jnp.cumsum(cnts, axis=0) - cnts
    bases = jnp.pad(tilebase, ((0, 0), (0, 16 - E))).astype(jnp.int32)
    nblocks = (poff[E] // BM).reshape(1)
    bidx = jnp.arange(NB, dtype=jnp.int32) * BM
    block_expert = jnp.sum(
        (poff[1:E + 1][None, :] <= bidx[:, None]).astype(jnp.int32), axis=1)
    block_expert = jnp.minimum(block_expert, E - 1)

    eid = jnp.concatenate([a1[:, 0], a2[:, 0]])

    xs, pos = _dispatch(eid, bases, x)
    ys = _grouped_ffn(block_expert, nblocks, xs, gate_up_proj, down_proj)

    pos_flat = pos.reshape(P)
    w1f, w2f = w1.reshape(N), w2.reshape(N)
    sh0 = _shared_half(xb, sh_gate_proj, sh_up_proj, sh_down_proj, 0)
    ymoe = _combine(ys, pos_flat, w1f, w2f, 0)
    out = _shared_final(xb, sh_gate_proj, sh_up_proj, sh_down_proj,
                        sh0, z, ymoe, None, 0)
    return out.reshape(B, S, H), aux[0, 0]
